# Initial kernel scaffold; baseline (speedup 1.0000x reference)
import jax
import jax.numpy as jnp
from jax import lax
from jax.experimental import pallas as pl
from jax.experimental.pallas import tpu as pltpu
from jax.experimental.pallas import tpu_sc as plsc

N = 100000
NUM_FEATURES = 9
VOCAB = 128
D = 128
HALF = D // 2
N_TILE = N // 16
CHUNK = 125
N_PAD = 100096
FEAT_PAD = 16
TBL_WORDS = NUM_FEATURES * VOCAB * HALF


def _body(feats_hbm, tables_hbm, out_hbm, tbl_v, feats_v, out_v):
    s = lax.axis_index("s")
    c = lax.axis_index("c")
    pltpu.sync_copy(tables_hbm.at[c], tbl_v)

    iota = lax.broadcasted_iota(jnp.int32, (16,), 0)
    ones = jnp.full((16,), 1, jnp.int32)

    def chunk_body(ci, carry):
        row0 = s * N_TILE + ci * CHUNK
        pltpu.sync_copy(feats_hbm.at[pl.ds(row0, 128), :], feats_v)
        for g in range(8):
            atomv = iota + g * 16
            mask = (iota < (CHUNK - 7 * 16)) if g == 7 else None
            bases = []
            for i in range(9):
                f = plsc.load_gather(
                    feats_v, [atomv, jnp.full((16,), i, jnp.int32)])
                bases.append(f * HALF + i * (VOCAB * HALF))

            def col_body(j, cc):
                jv = ones * j
                acc = plsc.load_gather(tbl_v, [bases[0] + jv])
                for i in range(1, 9):
                    acc = acc + plsc.load_gather(tbl_v, [bases[i] + jv])
                acc = acc * jnp.float32(1.0 / 3.0)
                plsc.store_scatter(out_v, [atomv, jv], acc, mask=mask)
                return cc

            lax.fori_loop(0, HALF, col_body, 0)
        pltpu.sync_copy(
            out_v, out_hbm.at[pl.ds(row0, CHUNK), pl.ds(c * HALF, HALF)])
        return carry

    lax.fori_loop(0, N_TILE // CHUNK, chunk_body, 0)


@jax.jit
def kernel(feats, tables):
    feats_p = jnp.pad(feats, ((0, N_PAD - N), (0, FEAT_PAD - NUM_FEATURES)))
    tables_r = tables.reshape(NUM_FEATURES, VOCAB, 2, HALF)
    tables_r = tables_r.transpose(2, 0, 1, 3).reshape(2, TBL_WORDS)
    run = pl.kernel(
        _body,
        out_type=jax.ShapeDtypeStruct((N, D), jnp.float32),
        mesh=plsc.VectorSubcoreMesh(
            core_axis_name="c", subcore_axis_name="s",
            num_cores=2, num_subcores=16),
        scratch_types=[
            pltpu.VMEM((TBL_WORDS,), jnp.float32),
            pltpu.VMEM((128, FEAT_PAD), jnp.int32),
            pltpu.VMEM((CHUNK, HALF), jnp.float32),
        ],
    )
    return run(feats_p, tables_r)


# speedup vs baseline: 1.2854x; 1.2854x over previous
import jax
import jax.numpy as jnp
from jax import lax
from jax.experimental import pallas as pl
from jax.experimental.pallas import tpu as pltpu
from jax.experimental.pallas import tpu_sc as plsc

N = 100000
NUM_FEATURES = 9
VOCAB = 128
D = 128
PAIRS = D // 2
N_TILES = 32
ATOMS_PER_TILE = 3136
CHUNK = 112
N_PAD = 100096
FEAT_PAD = 16
TBL_WORDS = NUM_FEATURES * VOCAB * PAIRS
LAST_FULL_CHUNKS = 24
TAIL_GROUPS = 6

SCALE = 1.0 / 3.0


def _emit_group(tbl_v, feats_v, out_v, chunk_base, g, iota, ones):
    atomv = iota + g * 16
    fidx = atomv * FEAT_PAD
    bases = []
    for i in range(9):
        f = plsc.load_gather(feats_v, [fidx + i])
        bases.append(f * PAIRS + i * (VOCAB * PAIRS))
    obase = atomv * D

    def pair_body(jb, cc):
        for u in range(4):
            j2 = jb * 4 + u
            j2v = ones * j2
            acc = plsc.bitcast(
                plsc.load_gather(tbl_v, [bases[0] + j2v]), jnp.bfloat16)
            for i in range(1, 9):
                acc = acc + plsc.bitcast(
                    plsc.load_gather(tbl_v, [bases[i] + j2v]), jnp.bfloat16)
            acc32 = plsc.bitcast(acc, jnp.int32)
            even = plsc.bitcast(lax.shift_left(acc32, 16), jnp.float32) * SCALE
            odd = plsc.bitcast(
                lax.bitwise_and(acc32, jnp.int32(-65536)), jnp.float32) * SCALE
            eidx = obase + j2 * 2
            plsc.store_scatter(out_v, [eidx], even)
            plsc.store_scatter(out_v, [eidx + 1], odd)
        return cc

    lax.fori_loop(0, PAIRS // 4, pair_body, 0)


def _emit_chunk(feats_hbm, out_hbm, tbl_v, feats_v, out_v, row0, ngroups,
                iota, ones):
    pltpu.sync_copy(
        feats_hbm.at[pl.ds(row0 * FEAT_PAD, CHUNK * FEAT_PAD)], feats_v)
    for g in range(ngroups):
        _emit_group(tbl_v, feats_v, out_v, row0, g, iota, ones)
    nrows = ngroups * 16
    pltpu.sync_copy(
        out_v.at[pl.ds(0, nrows * D)], out_hbm.at[pl.ds(row0 * D, nrows * D)])


def _body(feats_hbm, tables_hbm, out_hbm, tbl_v, feats_v, out_v):
    t = lax.axis_index("s") * 2 + lax.axis_index("c")
    pltpu.sync_copy(tables_hbm, tbl_v)

    iota = lax.broadcasted_iota(jnp.int32, (16,), 0)
    ones = jnp.full((16,), 1, jnp.int32)
    base = t * ATOMS_PER_TILE
    nchunks = jnp.where(t == N_TILES - 1, LAST_FULL_CHUNKS,
                        ATOMS_PER_TILE // CHUNK)

    def chunk_body(ci, carry):
        _emit_chunk(feats_hbm, out_hbm, tbl_v, feats_v, out_v,
                    base + ci * CHUNK, 7, iota, ones)
        return carry

    lax.fori_loop(0, nchunks, chunk_body, 0)

    @pl.when(t == N_TILES - 1)
    def _tail():
        _emit_chunk(feats_hbm, out_hbm, tbl_v, feats_v, out_v,
                    base + LAST_FULL_CHUNKS * CHUNK, TAIL_GROUPS, iota, ones)


@jax.jit
def kernel(feats, tables):
    feats_p = jnp.pad(
        feats, ((0, N_PAD - N), (0, FEAT_PAD - NUM_FEATURES))).reshape(-1)
    tbl = tables.astype(jnp.bfloat16).reshape(NUM_FEATURES * VOCAB, PAIRS, 2)
    tbl = lax.bitcast_convert_type(tbl, jnp.int32).reshape(-1)
    run = pl.kernel(
        _body,
        out_type=jax.ShapeDtypeStruct((N * D,), jnp.float32),
        mesh=plsc.VectorSubcoreMesh(
            core_axis_name="c", subcore_axis_name="s",
            num_cores=2, num_subcores=16),
        compiler_params=pltpu.CompilerParams(needs_layout_passes=False),
        scratch_types=[
            pltpu.VMEM((TBL_WORDS,), jnp.int32),
            pltpu.VMEM((CHUNK * FEAT_PAD,), jnp.int32),
            pltpu.VMEM((CHUNK * D,), jnp.float32),
        ],
    )
    return run(feats_p, tbl).reshape(N, D)


# speedup vs baseline: 3.3809x; 2.6301x over previous
import jax
import jax.numpy as jnp
from jax import lax
from jax.experimental import pallas as pl
from jax.experimental.pallas import tpu as pltpu
from jax.experimental.pallas import tpu_sc as plsc

N = 100000
NUM_FEATURES = 9
VOCAB = 128
D = 128
PAIRS = D // 2
ROW_STRIDE = PAIRS + 1
N_TILES = 32
ATOMS_PER_TILE = 3136
CHUNK = 112
N_PAD = 100096
FEAT_PAD = 16
TBL_WORDS = NUM_FEATURES * VOCAB * ROW_STRIDE
LAST_FULL_CHUNKS = 24
TAIL_GROUPS = 6

SCALE = 1.0 / 3.0


def _emit_group(tbl_v, feats_v, out_v, chunk_base, g, iota, ones):
    atomv = iota + g * 16
    fidx = atomv * FEAT_PAD
    bases = []
    for i in range(9):
        f = plsc.load_gather(feats_v, [fidx + i])
        bases.append(f * ROW_STRIDE + i * (VOCAB * ROW_STRIDE))
    obase = atomv * D

    def pair_body(jb, cc):
        for u in range(4):
            j2 = jb * 4 + u
            j2v = ones * j2
            g = [plsc.bitcast(plsc.load_gather(tbl_v, [bases[i] + j2v]),
                              jnp.bfloat16) for i in range(9)]
            t01, t23 = g[0] + g[1], g[2] + g[3]
            t45, t67 = g[4] + g[5], g[6] + g[7]
            acc = ((t01 + t23) + (t45 + t67)) + g[8]
            acc32 = plsc.bitcast(acc, jnp.int32)
            even = plsc.bitcast(lax.shift_left(acc32, 16), jnp.float32) * SCALE
            odd = plsc.bitcast(
                lax.bitwise_and(acc32, jnp.int32(-65536)), jnp.float32) * SCALE
            eidx = obase + j2 * 2
            plsc.store_scatter(out_v, [eidx], even)
            plsc.store_scatter(out_v, [eidx + 1], odd)
        return cc

    lax.fori_loop(0, PAIRS // 4, pair_body, 0)


def _emit_chunk(feats_hbm, out_hbm, tbl_v, feats_v, out_v, row0, ngroups,
                iota, ones):
    pltpu.sync_copy(
        feats_hbm.at[pl.ds(row0 * FEAT_PAD, CHUNK * FEAT_PAD)], feats_v)
    for g in range(ngroups):
        _emit_group(tbl_v, feats_v, out_v, row0, g, iota, ones)
    nrows = ngroups * 16
    pltpu.sync_copy(
        out_v.at[pl.ds(0, nrows * D)], out_hbm.at[pl.ds(row0 * D, nrows * D)])


def _body(feats_hbm, tables_hbm, out_hbm, tbl_v, feats_v, out_v):
    t = lax.axis_index("s") * 2 + lax.axis_index("c")
    pltpu.sync_copy(tables_hbm, tbl_v)

    iota = lax.broadcasted_iota(jnp.int32, (16,), 0)
    ones = jnp.full((16,), 1, jnp.int32)
    base = t * ATOMS_PER_TILE
    nchunks = jnp.where(t == N_TILES - 1, LAST_FULL_CHUNKS,
                        ATOMS_PER_TILE // CHUNK)

    def chunk_body(ci, carry):
        _emit_chunk(feats_hbm, out_hbm, tbl_v, feats_v, out_v,
                    base + ci * CHUNK, 7, iota, ones)
        return carry

    lax.fori_loop(0, nchunks, chunk_body, 0)

    @pl.when(t == N_TILES - 1)
    def _tail():
        _emit_chunk(feats_hbm, out_hbm, tbl_v, feats_v, out_v,
                    base + LAST_FULL_CHUNKS * CHUNK, TAIL_GROUPS, iota, ones)


@jax.jit
def kernel(feats, tables):
    feats_p = jnp.pad(
        feats, ((0, N_PAD - N), (0, FEAT_PAD - NUM_FEATURES))).reshape(-1)
    tbl = tables.astype(jnp.bfloat16).reshape(NUM_FEATURES * VOCAB, PAIRS, 2)
    tbl = lax.bitcast_convert_type(tbl, jnp.int32)
    tbl = jnp.pad(tbl, ((0, 0), (0, ROW_STRIDE - PAIRS))).reshape(-1)
    run = pl.kernel(
        _body,
        out_type=jax.ShapeDtypeStruct((N * D,), jnp.float32),
        mesh=plsc.VectorSubcoreMesh(
            core_axis_name="c", subcore_axis_name="s",
            num_cores=2, num_subcores=16),
        compiler_params=pltpu.CompilerParams(needs_layout_passes=False),
        scratch_types=[
            pltpu.VMEM((TBL_WORDS,), jnp.int32),
            pltpu.VMEM((CHUNK * FEAT_PAD,), jnp.int32),
            pltpu.VMEM((CHUNK * D,), jnp.float32),
        ],
    )
    return run(feats_p, tbl).reshape(N, D)


# speedup vs baseline: 4.9079x; 1.4516x over previous
import jax
import jax.numpy as jnp
from jax import lax
from jax.experimental import pallas as pl
from jax.experimental.pallas import tpu as pltpu
from jax.experimental.pallas import tpu_sc as plsc

N = 100000
NUM_FEATURES = 9
VOCAB = 128
D = 128
PAIRS = D // 2
N_TILES = 32
ATOMS_PER_TILE = 3136
CHUNK = 112
N_PAD = 100096
FEAT_PAD = 16
TBL_WORDS = NUM_FEATURES * VOCAB * PAIRS
LAST_FULL_CHUNKS = 24
TAIL_ATOMS = 96

SCALE = 1.0 / 3.0


def _emit_atom(tbl_v, feats_v, out_v, a, iota2):
    fv = feats_v[pl.ds(a * FEAT_PAD, FEAT_PAD)]
    bases = [fv[i] * PAIRS + i * (VOCAB * PAIRS) for i in range(9)]
    obase = a * D
    for c in range(4):
        g = [plsc.bitcast(tbl_v[pl.ds(bases[i] + c * 16, 16)], jnp.bfloat16)
             for i in range(9)]
        t01, t23 = g[0] + g[1], g[2] + g[3]
        t45, t67 = g[4] + g[5], g[6] + g[7]
        acc = ((t01 + t23) + (t45 + t67)) + g[8]
        acc32 = plsc.bitcast(acc, jnp.int32)
        even = plsc.bitcast(lax.shift_left(acc32, 16), jnp.float32) * SCALE
        odd = plsc.bitcast(
            lax.bitwise_and(acc32, jnp.int32(-65536)), jnp.float32) * SCALE
        seg = out_v.at[pl.ds(obase + c * 32, 32)]
        plsc.store_scatter(seg, [iota2], even)
        plsc.store_scatter(seg, [iota2 + 1], odd)


def _emit_chunk(feats_hbm, out_hbm, tbl_v, feats_v, out_v, row0, natoms,
                iota2):
    pltpu.sync_copy(
        feats_hbm.at[pl.ds(row0 * FEAT_PAD, CHUNK * FEAT_PAD)], feats_v)

    def atom_body(k, carry):
        _emit_atom(tbl_v, feats_v, out_v, k * 2, iota2)
        _emit_atom(tbl_v, feats_v, out_v, k * 2 + 1, iota2)
        return carry

    lax.fori_loop(0, natoms // 2, atom_body, 0)
    pltpu.sync_copy(
        out_v.at[pl.ds(0, natoms * D)],
        out_hbm.at[pl.ds(row0 * D, natoms * D)])


def _body(feats_hbm, tables_hbm, out_hbm, tbl_v, feats_v, out_v):
    t = lax.axis_index("s") * 2 + lax.axis_index("c")
    pltpu.sync_copy(tables_hbm, tbl_v)

    iota2 = lax.broadcasted_iota(jnp.int32, (16,), 0) * 2
    base = t * ATOMS_PER_TILE
    nchunks = jnp.where(t == N_TILES - 1, LAST_FULL_CHUNKS,
                        ATOMS_PER_TILE // CHUNK)

    def chunk_body(ci, carry):
        _emit_chunk(feats_hbm, out_hbm, tbl_v, feats_v, out_v,
                    base + ci * CHUNK, CHUNK, iota2)
        return carry

    lax.fori_loop(0, nchunks, chunk_body, 0)

    @pl.when(t == N_TILES - 1)
    def _tail():
        _emit_chunk(feats_hbm, out_hbm, tbl_v, feats_v, out_v,
                    base + LAST_FULL_CHUNKS * CHUNK, TAIL_ATOMS, iota2)


@jax.jit
def kernel(feats, tables):
    feats_p = jnp.pad(
        feats, ((0, N_PAD - N), (0, FEAT_PAD - NUM_FEATURES))).reshape(-1)
    tbl = tables.astype(jnp.bfloat16).reshape(NUM_FEATURES * VOCAB, PAIRS, 2)
    tbl = lax.bitcast_convert_type(tbl, jnp.int32).reshape(-1)
    run = pl.kernel(
        _body,
        out_type=jax.ShapeDtypeStruct((N * D,), jnp.float32),
        mesh=plsc.VectorSubcoreMesh(
            core_axis_name="c", subcore_axis_name="s",
            num_cores=2, num_subcores=16),
        compiler_params=pltpu.CompilerParams(needs_layout_passes=False),
        scratch_types=[
            pltpu.VMEM((TBL_WORDS,), jnp.int32),
            pltpu.VMEM((CHUNK * FEAT_PAD,), jnp.int32),
            pltpu.VMEM((CHUNK * D,), jnp.float32),
        ],
    )
    return run(feats_p, tbl).reshape(N, D)


# speedup vs baseline: 5.7220x; 1.1659x over previous
import jax
import jax.numpy as jnp
from jax import lax
from jax.experimental import pallas as pl
from jax.experimental.pallas import tpu as pltpu
from jax.experimental.pallas import tpu_sc as plsc

N = 100000
NUM_FEATURES = 9
VOCAB = 128
D = 128
PAIRS = D // 2
N_TILES = 32
ATOMS_PER_TILE = 3136
CHUNK = 64
N_PAD = 100096
FEAT_PAD = 16
TBL_WORDS = NUM_FEATURES * VOCAB * PAIRS
LAST_FULL_CHUNKS = 43
TAIL_ATOMS = 32
FWORDS = CHUNK * FEAT_PAD
OWORDS = CHUNK * D

SCALE = 1.0 / 3.0


def _emit_atom(tbl_v, fbuf, obuf, a, iota, iota2):
    fv = fbuf[pl.ds(a * FEAT_PAD, FEAT_PAD)] * PAIRS
    bases = [
        fv.at[jnp.full((16,), i, jnp.int32)].get(mode="promise_in_bounds")
        + i * (VOCAB * PAIRS)
        for i in range(9)
    ]
    obase = a * D
    for c in range(4):
        seg = iota + c * 16
        g = [plsc.bitcast(plsc.load_gather(tbl_v, [bases[i] + seg]),
                          jnp.bfloat16) for i in range(9)]
        t01, t23 = g[0] + g[1], g[2] + g[3]
        t45, t67 = g[4] + g[5], g[6] + g[7]
        acc = ((t01 + t23) + (t45 + t67)) + g[8]
        acc32 = plsc.bitcast(acc, jnp.int32)
        even = plsc.bitcast(lax.shift_left(acc32, 16), jnp.float32) * SCALE
        odd = plsc.bitcast(
            lax.bitwise_and(acc32, jnp.int32(-65536)), jnp.float32) * SCALE
        oseg = obuf.at[pl.ds(obase + c * 32, 32)]
        plsc.store_scatter(oseg, [iota2], even)
        plsc.store_scatter(oseg, [iota2 + 1], odd)


def _compute_chunk(tbl_v, fbuf, obuf, natoms, iota, iota2):
    def atom_body(k, carry):
        _emit_atom(tbl_v, fbuf, obuf, k * 2, iota, iota2)
        _emit_atom(tbl_v, fbuf, obuf, k * 2 + 1, iota, iota2)
        return carry

    lax.fori_loop(0, natoms // 2, atom_body, 0)


def _body(feats_hbm, tables_hbm, out_hbm, tbl_v, feats_v, out_v, fsem, osem):
    t = lax.axis_index("s") * 2 + lax.axis_index("c")
    pltpu.sync_copy(tables_hbm, tbl_v)

    iota = lax.broadcasted_iota(jnp.int32, (16,), 0)
    iota2 = iota * 2
    base = t * ATOMS_PER_TILE
    nchunks = jnp.where(t == N_TILES - 1, LAST_FULL_CHUNKS,
                        ATOMS_PER_TILE // CHUNK)

    def feats_dma(ci, b):
        row0 = base + ci * CHUNK
        return pltpu.make_async_copy(
            feats_hbm.at[pl.ds(row0 * FEAT_PAD, FWORDS)],
            feats_v.at[pl.ds(b * FWORDS, FWORDS)], fsem.at[b])

    feats_dma(0, 0).start()

    def chunk_body(ci, carry):
        b = lax.rem(ci, 2)
        @pl.when(ci + 1 < nchunks)
        def _pf():
            feats_dma(ci + 1, 1 - b).start()

        @pl.when(ci >= 2)
        def _drain():
            pltpu.make_async_copy(
                out_v.at[pl.ds(b * OWORDS, OWORDS)],
                out_hbm.at[pl.ds((base + (ci - 2) * CHUNK) * D, OWORDS)],
                osem.at[b]).wait()

        feats_dma(ci, b).wait()
        _compute_chunk(tbl_v, feats_v.at[pl.ds(b * FWORDS, FWORDS)],
                       out_v.at[pl.ds(b * OWORDS, OWORDS)],
                       CHUNK, iota, iota2)
        pltpu.async_copy(
            out_v.at[pl.ds(b * OWORDS, OWORDS)],
            out_hbm.at[pl.ds((base + ci * CHUNK) * D, OWORDS)],
            osem.at[b])
        return carry

    lax.fori_loop(0, nchunks, chunk_body, 0)

    for k in (2, 1):
        ci = nchunks - k
        b = lax.rem(ci, 2)
        pltpu.make_async_copy(
            out_v.at[pl.ds(b * OWORDS, OWORDS)],
            out_hbm.at[pl.ds((base + ci * CHUNK) * D, OWORDS)],
            osem.at[b]).wait()

    @pl.when(t == N_TILES - 1)
    def _tail():
        row0 = base + LAST_FULL_CHUNKS * CHUNK
        pltpu.sync_copy(
            feats_hbm.at[pl.ds(row0 * FEAT_PAD, TAIL_ATOMS * FEAT_PAD)],
            feats_v.at[pl.ds(0, TAIL_ATOMS * FEAT_PAD)])
        _compute_chunk(tbl_v, feats_v.at[pl.ds(0, FWORDS)],
                       out_v.at[pl.ds(0, OWORDS)], TAIL_ATOMS, iota, iota2)
        pltpu.sync_copy(
            out_v.at[pl.ds(0, TAIL_ATOMS * D)],
            out_hbm.at[pl.ds(row0 * D, TAIL_ATOMS * D)])


@jax.jit
def kernel(feats, tables):
    feats_p = jnp.pad(
        feats, ((0, N_PAD - N), (0, FEAT_PAD - NUM_FEATURES))).reshape(-1)
    tbl = tables.astype(jnp.bfloat16).reshape(NUM_FEATURES * VOCAB, PAIRS, 2)
    tbl = lax.bitcast_convert_type(tbl, jnp.int32).reshape(-1)
    run = pl.kernel(
        _body,
        out_type=jax.ShapeDtypeStruct((N * D,), jnp.float32),
        mesh=plsc.VectorSubcoreMesh(
            core_axis_name="c", subcore_axis_name="s",
            num_cores=2, num_subcores=16),
        compiler_params=pltpu.CompilerParams(needs_layout_passes=False),
        scratch_types=[
            pltpu.VMEM((TBL_WORDS,), jnp.int32),
            pltpu.VMEM((2 * FWORDS,), jnp.int32),
            pltpu.VMEM((2 * OWORDS,), jnp.float32),
            pltpu.SemaphoreType.DMA((2,)),
            pltpu.SemaphoreType.DMA((2,)),
        ],
    )
    return run(feats_p, tbl).reshape(N, D)


# speedup vs baseline: 5.7463x; 1.0043x over previous
import jax
import jax.numpy as jnp
from jax import lax
from jax.experimental import pallas as pl
from jax.experimental.pallas import tpu as pltpu
from jax.experimental.pallas import tpu_sc as plsc

N = 100000
NUM_FEATURES = 9
VOCAB = 128
D = 128
PAIRS = D // 2
N_TILES = 32
ATOMS_PER_TILE = 3136
CHUNK = 64
N_PAD = 100096
FEAT_PAD = 16
TBL_WORDS = NUM_FEATURES * VOCAB * PAIRS + 64
LAST_FULL_CHUNKS = 43
TAIL_ATOMS = 32
FWORDS = CHUNK * FEAT_PAD
OWORDS = CHUNK * D

SCALE = 1.0 / 3.0


def _emit_atom(tbl_v, fbuf, obuf, a, iota, iota2):
    fv = fbuf[pl.ds(a * FEAT_PAD, FEAT_PAD)] * PAIRS
    idx = [
        fv.at[jnp.full((16,), i, jnp.int32)].get(mode="promise_in_bounds")
        + iota
        for i in range(9)
    ]
    obase = a * D
    for c in range(4):
        g = [plsc.bitcast(
                plsc.load_gather(
                    tbl_v.at[pl.ds(i * (VOCAB * PAIRS) + c * 16,
                                   VOCAB * PAIRS)],
                    [idx[i]]),
                jnp.bfloat16) for i in range(9)]
        t01, t23 = g[0] + g[1], g[2] + g[3]
        t45, t67 = g[4] + g[5], g[6] + g[7]
        acc = ((t01 + t23) + (t45 + t67)) + g[8]
        acc32 = plsc.bitcast(acc, jnp.int32)
        even = plsc.bitcast(lax.shift_left(acc32, 16), jnp.float32) * SCALE
        odd = plsc.bitcast(
            lax.bitwise_and(acc32, jnp.int32(-65536)), jnp.float32) * SCALE
        oseg = obuf.at[pl.ds(obase + c * 32, 32)]
        plsc.store_scatter(oseg, [iota2], even)
        plsc.store_scatter(oseg, [iota2 + 1], odd)


def _compute_chunk(tbl_v, fbuf, obuf, natoms, iota, iota2):
    def atom_body(k, carry):
        _emit_atom(tbl_v, fbuf, obuf, k * 2, iota, iota2)
        _emit_atom(tbl_v, fbuf, obuf, k * 2 + 1, iota, iota2)
        return carry

    lax.fori_loop(0, natoms // 2, atom_body, 0)


def _body(feats_hbm, tables_hbm, out_hbm, tbl_v, feats_v, out_v, fsem, osem):
    t = lax.axis_index("s") * 2 + lax.axis_index("c")
    pltpu.sync_copy(tables_hbm, tbl_v)

    iota = lax.broadcasted_iota(jnp.int32, (16,), 0)
    iota2 = iota * 2
    base = t * ATOMS_PER_TILE
    nchunks = jnp.where(t == N_TILES - 1, LAST_FULL_CHUNKS,
                        ATOMS_PER_TILE // CHUNK)

    def feats_dma(ci, b):
        row0 = base + ci * CHUNK
        return pltpu.make_async_copy(
            feats_hbm.at[pl.ds(row0 * FEAT_PAD, FWORDS)],
            feats_v.at[pl.ds(b * FWORDS, FWORDS)], fsem.at[b])

    feats_dma(0, 0).start()

    def chunk_body(ci, carry):
        b = lax.rem(ci, 2)
        @pl.when(ci + 1 < nchunks)
        def _pf():
            feats_dma(ci + 1, 1 - b).start()

        @pl.when(ci >= 2)
        def _drain():
            pltpu.make_async_copy(
                out_v.at[pl.ds(b * OWORDS, OWORDS)],
                out_hbm.at[pl.ds((base + (ci - 2) * CHUNK) * D, OWORDS)],
                osem.at[b]).wait()

        feats_dma(ci, b).wait()
        _compute_chunk(tbl_v, feats_v.at[pl.ds(b * FWORDS, FWORDS)],
                       out_v.at[pl.ds(b * OWORDS, OWORDS)],
                       CHUNK, iota, iota2)
        pltpu.async_copy(
            out_v.at[pl.ds(b * OWORDS, OWORDS)],
            out_hbm.at[pl.ds((base + ci * CHUNK) * D, OWORDS)],
            osem.at[b])
        return carry

    lax.fori_loop(0, nchunks, chunk_body, 0)

    for k in (2, 1):
        ci = nchunks - k
        b = lax.rem(ci, 2)
        pltpu.make_async_copy(
            out_v.at[pl.ds(b * OWORDS, OWORDS)],
            out_hbm.at[pl.ds((base + ci * CHUNK) * D, OWORDS)],
            osem.at[b]).wait()

    @pl.when(t == N_TILES - 1)
    def _tail():
        row0 = base + LAST_FULL_CHUNKS * CHUNK
        pltpu.sync_copy(
            feats_hbm.at[pl.ds(row0 * FEAT_PAD, TAIL_ATOMS * FEAT_PAD)],
            feats_v.at[pl.ds(0, TAIL_ATOMS * FEAT_PAD)])
        _compute_chunk(tbl_v, feats_v.at[pl.ds(0, FWORDS)],
                       out_v.at[pl.ds(0, OWORDS)], TAIL_ATOMS, iota, iota2)
        pltpu.sync_copy(
            out_v.at[pl.ds(0, TAIL_ATOMS * D)],
            out_hbm.at[pl.ds(row0 * D, TAIL_ATOMS * D)])


@jax.jit
def kernel(feats, tables):
    feats_p = jnp.pad(
        feats, ((0, N_PAD - N), (0, FEAT_PAD - NUM_FEATURES))).reshape(-1)
    tbl = tables.astype(jnp.bfloat16).reshape(NUM_FEATURES * VOCAB, PAIRS, 2)
    tbl = lax.bitcast_convert_type(tbl, jnp.int32).reshape(-1)
    tbl = jnp.pad(tbl, (0, TBL_WORDS - tbl.shape[0]))
    run = pl.kernel(
        _body,
        out_type=jax.ShapeDtypeStruct((N * D,), jnp.float32),
        mesh=plsc.VectorSubcoreMesh(
            core_axis_name="c", subcore_axis_name="s",
            num_cores=2, num_subcores=16),
        compiler_params=pltpu.CompilerParams(needs_layout_passes=False),
        scratch_types=[
            pltpu.VMEM((TBL_WORDS,), jnp.int32),
            pltpu.VMEM((2 * FWORDS,), jnp.int32),
            pltpu.VMEM((2 * OWORDS,), jnp.float32),
            pltpu.SemaphoreType.DMA((2,)),
            pltpu.SemaphoreType.DMA((2,)),
        ],
    )
    return run(feats_p, tbl).reshape(N, D)
